# Initial kernel scaffold; baseline (speedup 1.0000x reference)
import jax
import jax.numpy as jnp
from jax.experimental import pallas as pl


def kernel(pos, batch, params):
    raise NotImplementedError("write your pallas kernel here")


# speedup vs baseline: 9.7354x; 9.7354x over previous
import functools
import math

import jax
import jax.numpy as jnp
from jax import lax
from jax.experimental import pallas as pl
from jax.experimental.pallas import tpu as pltpu

_BN = 1.0 / math.sqrt(1.0 + 1e-5)
_NB = 8


def _elu(x):
    return jnp.where(x > 0, x, jnp.exp(jnp.minimum(x, 0.0)) - 1.0)


def _knn_kernel(posr_ref, batr_ref, posT_ref, batT_ref, out_ref, D_ref,
                *, k, dil, K, n):
    R = posr_ref.shape[0]
    px = posr_ref[:, 0:1]
    py = posr_ref[:, 1:2]
    pz = posr_ref[:, 2:3]
    qx = posT_ref[0:1, :]
    qy = posT_ref[1:2, :]
    qz = posT_ref[2:3, :]
    same = batr_ref[...] == batT_ref[...]
    d = (px - qx) ** 2 + (py - qy) ** 2 + (pz - qz) ** 2
    D_ref[...] = jnp.where(same, d, jnp.inf)
    cnt = jnp.sum(same.astype(jnp.int32), axis=1, keepdims=True)
    pcol = (lax.broadcasted_iota(jnp.int32, (R, k), 1) * dil) % cnt
    iota = lax.broadcasted_iota(jnp.int32, (R, n), 1)

    def pass_body(t, out):
        D = D_ref[...]
        m = jnp.min(D, axis=1, keepdims=True)
        idx = jnp.min(jnp.where(D <= m, iota, n), axis=1, keepdims=True)
        D_ref[...] = jnp.where(iota == idx, jnp.inf, D)
        return jnp.where(pcol == t, idx, out)

    out_ref[...] = lax.fori_loop(0, K, pass_body,
                                 jnp.zeros((R, k), jnp.int32))


def _knn(pos, batch, k, dil):
    n = pos.shape[0]
    R = 256
    while n % R:
        R //= 2
    K = (k - 1) * dil + 1
    body = functools.partial(_knn_kernel, k=k, dil=dil, K=K, n=n)
    return pl.pallas_call(
        body,
        grid=(n // R,),
        in_specs=[
            pl.BlockSpec((R, 3), lambda i: (i, 0)),
            pl.BlockSpec((R, 1), lambda i: (i, 0)),
            pl.BlockSpec((3, n), lambda i: (0, 0)),
            pl.BlockSpec((1, n), lambda i: (0, 0)),
        ],
        out_specs=pl.BlockSpec((R, k), lambda i: (i, 0)),
        out_shape=jax.ShapeDtypeStruct((n, k), jnp.int32),
        scratch_shapes=[pltpu.VMEM((R, n), jnp.float32)],
    )(pos, batch.reshape(n, 1), pos.T, batch.reshape(1, n))


def _fps_kernel(posT_ref, batT_ref, s_ref, sel_ref, dist_ref, *, nb, n):
    X = posT_ref[0:1, :]
    Y = posT_ref[1:2, :]
    Z = posT_ref[2:3, :]
    bid = lax.broadcasted_iota(jnp.int32, (nb, n), 0)
    iota = lax.broadcasted_iota(jnp.int32, (nb, n), 1)
    maskb = bid == batT_ref[...]
    first = jnp.min(jnp.where(maskb, iota, n), axis=1, keepdims=True)
    first = jnp.where(first == n, 0, first)
    oh = iota == first
    fx = jnp.sum(jnp.where(oh, X, 0.0), axis=1, keepdims=True)
    fy = jnp.sum(jnp.where(oh, Y, 0.0), axis=1, keepdims=True)
    fz = jnp.sum(jnp.where(oh, Z, 0.0), axis=1, keepdims=True)
    dist_ref[...] = (X - fx) ** 2 + (Y - fy) ** 2 + (Z - fz) ** 2
    sel_ref[0:1, :] = first.reshape(1, nb)
    T = jnp.max(s_ref[...])

    def body(t, carry):
        dist = dist_ref[...]
        dm = jnp.where(maskb, dist, -jnp.inf)
        m = jnp.max(dm, axis=1, keepdims=True)
        j = jnp.min(jnp.where(dm >= m, iota, n), axis=1, keepdims=True)
        sel_ref[pl.ds(t, 1), :] = j.reshape(1, nb)
        ohj = iota == j
        px = jnp.sum(jnp.where(ohj, X, 0.0), axis=1, keepdims=True)
        py = jnp.sum(jnp.where(ohj, Y, 0.0), axis=1, keepdims=True)
        pz = jnp.sum(jnp.where(ohj, Z, 0.0), axis=1, keepdims=True)
        nd = (X - px) ** 2 + (Y - py) ** 2 + (Z - pz) ** 2
        dist_ref[...] = jnp.minimum(dist, nd)
        return carry

    lax.fori_loop(1, T, body, 0)


def _fps(pos, batch, ratio):
    n = pos.shape[0]
    nb = _NB
    cap = int(math.ceil(ratio * n))
    table = jnp.asarray([int(math.ceil(ratio * c)) for c in range(n + 1)],
                        dtype=jnp.int32)
    cnt = jnp.sum(batch[None, :] == jnp.arange(nb)[:, None], axis=1)
    s = jnp.where(cnt > 0, table[cnt], 0).astype(jnp.int32)
    sel = pl.pallas_call(
        functools.partial(_fps_kernel, nb=nb, n=n),
        out_shape=jax.ShapeDtypeStruct((cap, nb), jnp.int32),
        scratch_shapes=[pltpu.VMEM((nb, n), jnp.float32)],
    )(pos.T, batch.reshape(1, n), s.reshape(nb, 1))
    selT = sel.T
    off = jnp.concatenate([jnp.zeros((1,), s.dtype), jnp.cumsum(s)[:-1]])
    t = jnp.arange(cap)
    posn = jnp.where(t[None, :] < s[:, None], off[:, None] + t[None, :], n)
    idx = jnp.zeros((n,), jnp.int32).at[posn.reshape(-1)].set(
        selT.reshape(-1), mode='drop')
    valid = jnp.arange(n) < s.sum()
    return idx, valid


def _xconv_kernel(*refs, k, cd, cin, dm, cout):
    if cin > 0:
        (rel_ref, xg_ref, w1T_ref, b1_ref, w2T_ref, b2_ref, m2wT_ref,
         m2b_ref, cawT_ref, cab_ref, cbwT_ref, cbb_ref, dwhT_ref, dwbh_ref,
         dwxT_ref, dwbx_ref, WhT_ref, WxT_ref, linb_ref, out_ref) = refs
    else:
        (rel_ref, w1T_ref, b1_ref, w2T_ref, b2_ref, m2wT_ref,
         m2b_ref, cawT_ref, cab_ref, cbwT_ref, cbb_ref, dwhT_ref, dwbh_ref,
         WhT_ref, linb_ref, out_ref) = refs
    Bk = rel_ref.shape[0]
    B = Bk // k
    rel = rel_ref[...]
    h = _elu(jnp.dot(rel, w1T_ref[...],
                     preferred_element_type=jnp.float32) + b1_ref[...]) * _BN
    h = _elu(jnp.dot(h, w2T_ref[...],
                     preferred_element_type=jnp.float32) + b2_ref[...]) * _BN
    h3 = h.reshape(B, k, cd)
    r3 = rel.reshape(B, k, 3)
    if cin > 0:
        xg3 = xg_ref[...].reshape(B, k, cin)
    tacc = jnp.zeros((B, k * k), jnp.float32)
    for l in range(k):
        tacc += jnp.dot(r3[:, l, :], m2wT_ref[3 * l:3 * l + 3, :],
                        preferred_element_type=jnp.float32)
    t = _elu(tacc + m2b_ref[...]) * _BN
    y_h = jnp.zeros((B, k, cd), jnp.float32)
    if cin > 0:
        y_x = jnp.zeros((B, k, cin), jnp.float32)
    for l in range(k):
        tl = t[:, l * k:(l + 1) * k]
        ta = _elu(jnp.dot(tl, cawT_ref[l],
                          preferred_element_type=jnp.float32)
                  + cab_ref[l:l + 1, :]) * _BN
        tb = (jnp.dot(ta, cbwT_ref[l],
                      preferred_element_type=jnp.float32)
              + cbb_ref[l:l + 1, :]) * _BN
        y_h += tb[:, :, None] * h3[:, l, :][:, None, :]
        if cin > 0:
            y_x += tb[:, :, None] * xg3[:, l, :][:, None, :]
    acc = jnp.zeros((B, cout), jnp.float32) + linb_ref[...]
    for j in range(dm):
        o_h = jnp.zeros((B, cd), jnp.float32) + dwbh_ref[j:j + 1, :]
        for l in range(k):
            o_h += y_h[:, l, :] * dwhT_ref[j * k + l:j * k + l + 1, :]
        acc += jnp.dot(o_h, WhT_ref[j * cd:(j + 1) * cd, :],
                       preferred_element_type=jnp.float32)
        if cin > 0:
            o_x = jnp.zeros((B, cin), jnp.float32) + dwbx_ref[j:j + 1, :]
            for l in range(k):
                o_x += y_x[:, l, :] * dwxT_ref[j * k + l:j * k + l + 1, :]
            acc += jnp.dot(o_x, WxT_ref[j * cin:(j + 1) * cin, :],
                           preferred_element_type=jnp.float32)
    out_ref[...] = jnp.maximum(acc, 0.0)


def _xconv(x, pos, col, p, k):
    n = pos.shape[0]
    cd = p['m1b1'].shape[0]
    cout = p['lin_b'].shape[0]
    cin = 0 if x is None else x.shape[1]
    c = cin + cd
    dm = int(math.ceil(cout / c))
    cf = col.reshape(-1)
    rel = pos[cf] - jnp.repeat(pos, k, axis=0)

    w1T = p['m1w1'].T
    b1 = p['m1b1'].reshape(1, cd)
    w2T = p['m1w2'].T
    b2 = p['m1b2'].reshape(1, cd)
    m2wT = p['m2w'].T
    m2b = p['m2b'].reshape(1, k * k)
    cawT = jnp.transpose(p['m2ca_w'], (0, 2, 1))
    cbwT = jnp.transpose(p['m2cb_w'], (0, 2, 1))
    cab = p['m2ca_b']
    cbb = p['m2cb_b']
    dwT = jnp.transpose(p['dw_w'], (1, 2, 0))
    dwhT = dwT[:, :, :cd].reshape(dm * k, cd)
    dwbT = p['dw_b'].T
    dwbh = dwbT[:, :cd]
    lin3 = p['lin_w'].reshape(cout, c, dm)
    WhT = jnp.transpose(lin3[:, :cd, :], (2, 1, 0)).reshape(dm * cd, cout)
    linb = p['lin_b'].reshape(1, cout)

    B = 256
    while n % B:
        B //= 2
    grid = (n // B,)
    full = lambda a: pl.BlockSpec(a.shape, lambda i: (0,) * a.ndim)
    ops = [rel]
    specs = [pl.BlockSpec((B * k, 3), lambda i: (i, 0))]
    if cin > 0:
        xg = x[cf]
        dwxT = dwT[:, :, cd:].reshape(dm * k, cin)
        dwbx = dwbT[:, cd:]
        WxT = jnp.transpose(lin3[:, cd:, :], (2, 1, 0)).reshape(dm * cin, cout)
        ops += [xg]
        specs += [pl.BlockSpec((B * k, cin), lambda i: (i, 0))]
        wts = [w1T, b1, w2T, b2, m2wT, m2b, cawT, cab, cbwT, cbb,
               dwhT, dwbh, dwxT, dwbx, WhT, WxT, linb]
    else:
        wts = [w1T, b1, w2T, b2, m2wT, m2b, cawT, cab, cbwT, cbb,
               dwhT, dwbh, WhT, linb]
    ops += wts
    specs += [full(w) for w in wts]
    body = functools.partial(_xconv_kernel, k=k, cd=cd, cin=cin, dm=dm,
                             cout=cout)
    return pl.pallas_call(
        body,
        grid=grid,
        in_specs=specs,
        out_specs=pl.BlockSpec((B, cout), lambda i: (i, 0)),
        out_shape=jax.ShapeDtypeStruct((n, cout), jnp.float32),
    )(*ops)


def _head_kernel(x_ref, b2T_ref, w1T_ref, b1_ref, w2T_ref, b2_ref,
                 w3T_ref, b3_ref, out_ref, *, nb):
    n = x_ref.shape[0]
    oh = (lax.broadcasted_iota(jnp.int32, (nb, n), 0)
          == b2T_ref[...]).astype(jnp.float32)
    s = jnp.dot(oh, x_ref[...], preferred_element_type=jnp.float32)
    cnt = jnp.sum(oh, axis=1, keepdims=True)
    xm = s / jnp.maximum(cnt, 1.0)
    h = jnp.maximum(jnp.dot(xm, w1T_ref[...],
                            preferred_element_type=jnp.float32)
                    + b1_ref[...], 0.0)
    h = jnp.maximum(jnp.dot(h, w2T_ref[...],
                            preferred_element_type=jnp.float32)
                    + b2_ref[...], 0.0)
    o = jnp.dot(h, w3T_ref[...],
                preferred_element_type=jnp.float32) + b3_ref[...]
    sh = o - jnp.max(o, axis=1, keepdims=True)
    out_ref[...] = sh - jnp.log(jnp.sum(jnp.exp(sh), axis=1, keepdims=True))


def _head(x, b2, params):
    n = x.shape[0]
    nb = _NB
    return pl.pallas_call(
        functools.partial(_head_kernel, nb=nb),
        out_shape=jax.ShapeDtypeStruct((nb, 40), jnp.float32),
    )(x, b2.reshape(1, n).astype(jnp.int32),
      params['l1w'].T, params['l1b'].reshape(1, -1),
      params['l2w'].T, params['l2b'].reshape(1, -1),
      params['l3w'].T, params['l3b'].reshape(1, -1))


def kernel(pos, batch, params):
    nb = _NB
    batch = batch.astype(jnp.int32)
    col1 = _knn(pos, batch, 8, 1)
    idx1, v1 = _fps(pos, batch, 0.375)
    b1 = jnp.where(v1, batch[idx1], nb)
    p1 = pos[idx1]
    col2 = _knn(p1, b1, 12, 2)
    idx2, v2 = _fps(p1, b1, 0.334)
    b2 = jnp.where(v2, b1[idx2], nb)
    p2 = p1[idx2]
    col34 = _knn(p2, b2, 16, 2)
    x = _xconv(None, pos, col1, params['c1'], 8)
    x = x[idx1]
    x = _xconv(x, p1, col2, params['c2'], 12)
    x = x[idx2]
    x = _xconv(x, p2, col34, params['c3'], 16)
    x = _xconv(x, p2, col34, params['c4'], 16)
    return _head(x, b2, params)
